# baseline (speedup 1.0000x reference)
import jax
import jax.numpy as jnp
from jax.experimental import pallas as pl
from jax.experimental.pallas import tpu as pltpu

_NEG = -1.0e30


def _fused_kernel(rep_ref, mask_ref, w_ref, b_ref, wm_ref, bm_ref,
                  valid_ref, out_ref, sv_ref, ev_ref):
    j = pl.program_id(1)
    ti, s = out_ref.shape

    @pl.when(j == 0)
    def _():
        x = rep_ref[0]
        logits = jnp.dot(x, w_ref[...], preferred_element_type=jnp.float32)
        logits = logits + b_ref[...]
        score = jnp.dot(logits, wm_ref[...], preferred_element_type=jnp.float32)
        m = mask_ref[0].T != 0
        s_ok = m & (logits[:, 0:1] <= logits[:, 1:2])
        e_ok = m & (logits[:, 2:3] <= logits[:, 3:4])
        sv_ref[...] = jnp.where(s_ok, score[:, 0:1], _NEG)
        ev = jnp.where(e_ok, score[:, 1:2], _NEG)
        ev_ref[...] = ev.T

    @pl.when(j > 0)
    def _():
        base = (j - 1) * ti
        svb = sv_ref[pl.ds(base, ti), :]
        pair = (svb + ev_ref[...]) + bm_ref[...]
        rowv = jax.lax.broadcasted_iota(jnp.int32, (ti, 1), 0) + base
        colv = jax.lax.broadcasted_iota(jnp.int32, (ti, s), 1)
        valid = (colv >= rowv) & (pair > 0)
        valid_ref[...] = jnp.where(valid, 1, 0).astype(jnp.int8)
        out_ref[...] = jnp.where(valid, pair, 0.0)


def kernel(rep, mask, Ws, bs, We, be, Wm, bm):
    B, S, H = rep.shape
    N = B * S
    mask3 = mask.reshape(B, 1, S)
    w4 = jnp.concatenate([Ws, We], axis=1)
    b4 = jnp.concatenate([bs, be]).reshape(1, 4)
    z2 = jnp.zeros_like(Wm[:2, :1])
    wmm = jnp.concatenate(
        [jnp.concatenate([Wm[:2, :1], z2], axis=0),
         jnp.concatenate([z2, Wm[2:, :1]], axis=0)], axis=1)
    bm1 = bm.reshape(1, 1)

    TI = 512
    RB = S // TI
    valid2, masked2 = pl.pallas_call(
        _fused_kernel,
        grid=(B, RB + 1),
        in_specs=[
            pl.BlockSpec((1, S, H), lambda b, j: (b, 0, 0)),
            pl.BlockSpec((1, 1, S), lambda b, j: (b, 0, 0)),
            pl.BlockSpec((H, 4), lambda b, j: (0, 0)),
            pl.BlockSpec((1, 4), lambda b, j: (0, 0)),
            pl.BlockSpec((4, 2), lambda b, j: (0, 0)),
            pl.BlockSpec((1, 1), lambda b, j: (0, 0)),
        ],
        out_specs=[
            pl.BlockSpec((TI, S), lambda b, j: (b * RB + jnp.maximum(j - 1, 0), 0)),
            pl.BlockSpec((TI, S), lambda b, j: (b * RB + jnp.maximum(j - 1, 0), 0)),
        ],
        out_shape=[
            jax.ShapeDtypeStruct((N, S), jnp.int8),
            jax.ShapeDtypeStruct((N, S), jnp.float32),
        ],
        scratch_shapes=[
            pltpu.VMEM((S, 1), jnp.float32),
            pltpu.VMEM((1, S), jnp.float32),
        ],
        compiler_params=pltpu.CompilerParams(
            dimension_semantics=("arbitrary", "arbitrary"),
        ),
        name="span_fused",
    )(rep, mask3, w4, b4, wmm, bm1)

    return (valid2.view(jnp.bool_).reshape(B, S, S),
            masked2.reshape(B, S, S))
